# Initial kernel scaffold; baseline (speedup 1.0000x reference)
import jax
import jax.numpy as jnp
from jax.experimental import pallas as pl


def kernel(x):
    raise NotImplementedError("write your pallas kernel here")


# speedup vs baseline: 6.0704x; 6.0704x over previous
import functools

import jax
import jax.numpy as jnp
from jax import lax
from jax.experimental import pallas as pl
from jax.experimental.pallas import tpu as pltpu
from jax.experimental.pallas import tpu_sc as plsc

_MASK_RATIO = 0.15
_B, _L, _D = 16, 2048, 1024
_LEN_KEEP = int(_L * (1 - _MASK_RATIO))
_R = _B * _LEN_KEEP

_NC, _NS = 2, 16
_NW = _NC * _NS
_W = 40
_GCH = _R // _W
_SLOTS = -(-_GCH // _NW)


def _index_arrays():
    noise = jax.random.uniform(
        jax.random.fold_in(jax.random.key(0), 1), (_B, _L), dtype=jnp.float32)
    shuf = jnp.argsort(noise, axis=1).astype(jnp.int32)
    keep = shuf[:, :_LEN_KEEP]
    gidx = keep + (jnp.arange(_B, dtype=jnp.int32) * _L)[:, None]
    flat = gidx.reshape(-1)
    return jnp.pad(flat, (0, _SLOTS * _NW * _W - _R)), shuf.reshape(-1)


def _body(x_hbm, gidx_hbm, shuf_hbm, out_hbm, restore_hbm,
          idx_v, rows_v, shuf_v, rest_v, sem0, sem1):
    cid = lax.axis_index("c")
    sid = lax.axis_index("s")
    wid = sid * _NC + cid
    sems = (sem0, sem1)

    @pl.when(wid < _B)
    def _():
        pltpu.sync_copy(shuf_hbm.at[pl.ds(wid * _L, _L)], shuf_v)

        def it(i, carry):
            tgt = shuf_v[pl.ds(i * 16, 16)]
            vals = lax.iota(jnp.int32, 16) + i * 16
            plsc.store_scatter(rest_v, [tgt], vals)
            return carry

        lax.fori_loop(0, _L // 16, it, 0)
        pltpu.sync_copy(rest_v, restore_hbm.at[pl.ds(wid * _L, _L)])

    def start(c):
        b = c % 2
        g = c * _NW + wid
        pltpu.sync_copy(gidx_hbm.at[pl.ds(g * _W, _W)], idx_v.at[b])
        return pltpu.async_copy(x_hbm.at[idx_v.at[b]], rows_v.at[b], sems[b])

    handle = start(0)
    for c in range(_SLOTS):
        nxt = start(c + 1) if c + 1 < _SLOTS else None
        handle.wait()
        g = c * _NW + wid

        @pl.when(g < _GCH)
        def _(c=c, g=g):
            pltpu.sync_copy(rows_v.at[c % 2], out_hbm.at[pl.ds(g * _W, _W)])

        handle = nxt


@functools.cache
def _sc_gather():
    return pl.kernel(
        _body,
        out_type=(
            jax.ShapeDtypeStruct((_R, _D), jnp.float32),
            jax.ShapeDtypeStruct((_B * _L,), jnp.int32),
        ),
        mesh=plsc.VectorSubcoreMesh(core_axis_name="c", subcore_axis_name="s"),
        compiler_params=pltpu.CompilerParams(needs_layout_passes=False),
        scratch_types=(
            pltpu.VMEM((2, _W), jnp.int32),
            pltpu.VMEM((2, _W, _D), jnp.float32),
            pltpu.VMEM((_L,), jnp.int32),
            pltpu.VMEM((_L,), jnp.int32),
            pltpu.SemaphoreType.DMA,
            pltpu.SemaphoreType.DMA,
        ),
    )


def kernel(x):
    gidx, shuf = _index_arrays()
    x_flat = x.reshape(_B * _L, _D)
    out_flat, restore = _sc_gather()(x_flat, gidx, shuf)
    return (out_flat.reshape(_B, _LEN_KEEP, _D), x,
            restore.reshape(_B, _L))
